# baseline (speedup 1.0000x reference)
import functools

import jax
import jax.numpy as jnp
from jax import lax
from jax.experimental import pallas as pl
from jax.experimental.pallas import tpu as pltpu
from jax.experimental.pallas import tpu_sc as plsc

B = 16384
F = 26
K = 16
NC = 2
NS = 16
NW = NC * NS
ITEMS_W = B // NW
ROWS_W = ITEMS_W * F
GI = 16
ROWS_G = GI * F
NG = ITEMS_W // GI
NSUB = 4
SUB = ROWS_G // NSUB

_mesh = plsc.VectorSubcoreMesh(core_axis_name="c", subcore_axis_name="s")


@functools.partial(
    pl.kernel,
    out_type=jax.ShapeDtypeStruct((B,), jnp.float32),
    mesh=_mesh,
    compiler_params=pltpu.CompilerParams(
        needs_layout_passes=False, use_tc_tiling_on_sc=False),
    scratch_types=[
        pltpu.VMEM((NSUB * NG, SUB), jnp.int32),
        pltpu.VMEM((2, ROWS_G, K), jnp.float32),
        pltpu.VMEM((ROWS_G,), jnp.float32),
        pltpu.VMEM((ROWS_G,), jnp.float32),
        pltpu.VMEM((ITEMS_W,), jnp.float32),
        pltpu.VMEM((16,), jnp.float32),
        pltpu.SemaphoreType.DMA,
        pltpu.SemaphoreType.DMA,
    ],
)
def _fm_sc(x_hbm, emb_hbm, fc_hbm, wb_hbm, out_hbm,
           idx_v, rows_v, fcv0, fcv1, outbuf, wbv, sem0, sem1):
    wid = lax.axis_index("s") * NC + lax.axis_index("c")

    pltpu.sync_copy(x_hbm.at[wid], idx_v)
    pltpu.sync_copy(wb_hbm, wbv)
    wb16 = wbv[:]
    w = wb16[0]
    bias = wb16[1]

    lanes = lax.iota(jnp.int32, 16)

    _dn = lax.GatherDimensionNumbers(
        offset_dims=(), collapsed_slice_dims=(0,), start_index_map=(0,))

    def allsum16(v):
        for sh in (8, 4, 2, 1):
            perm = lax.gather(v, (lanes ^ sh)[:, None], dimension_numbers=_dn,
                              slice_sizes=(1,),
                              mode=lax.GatherScatterMode.PROMISE_IN_BOUNDS)
            v = v + perm
        return v

    def fire(g, buf):
        rbuf = rows_v.at[buf]
        fbuf = fcv0 if buf == 0 else fcv1
        for i in range(NSUB):
            j = g * NSUB + i
            pltpu.async_copy(emb_hbm.at[idx_v.at[j]],
                             rbuf.at[pl.ds(i * SUB, SUB)], sem0 if buf == 0 else sem1)
            pltpu.async_copy(fc_hbm.at[idx_v.at[j]],
                             fbuf.at[pl.ds(i * SUB, SUB)], sem0 if buf == 0 else sem1)

    def drain(g, buf):
        rbuf = rows_v.at[buf]
        fbuf = fcv0 if buf == 0 else fcv1
        for i in range(NSUB):
            j = g * NSUB + i
            pltpu.make_async_copy(emb_hbm.at[idx_v.at[j]],
                                  rbuf.at[pl.ds(i * SUB, SUB)],
                                  sem0 if buf == 0 else sem1).wait()
            pltpu.make_async_copy(fc_hbm.at[idx_v.at[j]],
                                  fbuf.at[pl.ds(i * SUB, SUB)],
                                  sem0 if buf == 0 else sem1).wait()

    def compute(g, buf):
        rbuf = rows_v.at[buf]

        def item_body(b, pv):
            r0 = b * F
            s = rbuf[r0, :]
            ss = s * s
            for f in range(1, F):
                e = rbuf[r0 + f, :]
                s = s + e
                ss = ss + e * e
            pair = 0.5 * allsum16(s * s - ss)
            return jnp.where(lanes == b, pair, pv)

        pairvec = lax.fori_loop(0, GI, item_body, jnp.zeros((16,), jnp.float32))

        fbuf = fcv0 if buf == 0 else fcv1
        rowbase = lanes * F
        fcs = jnp.zeros((16,), jnp.float32)
        for f in range(F):
            fcs = fcs + plsc.load_gather(fbuf, [rowbase + f])

        z = pairvec + fcs * w + bias
        outbuf[pl.ds(g * GI, GI)] = 1.0 / (1.0 + jnp.exp(-z))

    fire(0, 0)

    def pair_body(p, _):
        g0 = 2 * p
        fire(g0 + 1, 1)
        drain(g0, 0)
        compute(g0, 0)

        @pl.when(g0 + 2 < NG)
        def _():
            fire(g0 + 2, 0)

        drain(g0 + 1, 1)
        compute(g0 + 1, 1)
        return 0

    lax.fori_loop(0, NG // 2, pair_body, 0)

    pltpu.sync_copy(outbuf, out_hbm.at[pl.ds(wid * ITEMS_W, ITEMS_W)])


def kernel(x, emb_table, fc_table, lin_w, lin_b):
    x3 = x.reshape(NW, NSUB * NG, SUB)
    wb = jnp.zeros((16,), jnp.float32)
    wb = wb.at[0].set(lin_w[0, 0]).at[1].set(lin_b[0])
    out = _fm_sc(x3, emb_table, fc_table.reshape(-1), wb)
    return out.reshape(B, 1)


# speedup vs baseline: 1.0012x; 1.0012x over previous
import functools

import jax
import jax.numpy as jnp
from jax import lax
from jax.experimental import pallas as pl
from jax.experimental.pallas import tpu as pltpu
from jax.experimental.pallas import tpu_sc as plsc

B = 16384
F = 26
K = 16
NC = 2
NS = 16
NW = NC * NS
ITEMS_W = B // NW
ROWS_W = ITEMS_W * F
GI = 16
ROWS_G = GI * F
CI = 64
ROWS_C = CI * F
NCH = ITEMS_W // CI
GPC = CI // GI

_mesh = plsc.VectorSubcoreMesh(core_axis_name="c", subcore_axis_name="s")


@functools.partial(
    pl.kernel,
    out_type=jax.ShapeDtypeStruct((B,), jnp.float32),
    mesh=_mesh,
    compiler_params=pltpu.CompilerParams(
        needs_layout_passes=False, use_tc_tiling_on_sc=False),
    scratch_types=[
        pltpu.VMEM((NCH, ROWS_C), jnp.int32),
        pltpu.VMEM((2, ROWS_C, K), jnp.float32),
        pltpu.VMEM((ROWS_C,), jnp.float32),
        pltpu.VMEM((ROWS_C,), jnp.float32),
        pltpu.VMEM((ITEMS_W,), jnp.float32),
        pltpu.VMEM((16,), jnp.float32),
        pltpu.SemaphoreType.DMA,
        pltpu.SemaphoreType.DMA,
    ],
)
def _fm_sc(x_hbm, emb_hbm, fc_hbm, wb_hbm, out_hbm,
           idx_v, rows_v, fcv0, fcv1, outbuf, wbv, sem0, sem1):
    wid = lax.axis_index("s") * NC + lax.axis_index("c")

    pltpu.sync_copy(x_hbm.at[wid], idx_v)
    pltpu.sync_copy(wb_hbm, wbv)
    wb16 = wbv[:]
    w = wb16[0]
    bias = wb16[1]

    lanes = lax.iota(jnp.int32, 16)

    _dn = lax.GatherDimensionNumbers(
        offset_dims=(), collapsed_slice_dims=(0,), start_index_map=(0,))

    def allsum16(v):
        for sh in (8, 4, 2, 1):
            perm = lax.gather(v, (lanes ^ sh)[:, None], dimension_numbers=_dn,
                              slice_sizes=(1,),
                              mode=lax.GatherScatterMode.PROMISE_IN_BOUNDS)
            v = v + perm
        return v

    def fire(c, buf):
        rbuf = rows_v.at[buf]
        fbuf = fcv0 if buf == 0 else fcv1
        sem = sem0 if buf == 0 else sem1
        pltpu.async_copy(emb_hbm.at[idx_v.at[c]], rbuf, sem)
        pltpu.async_copy(fc_hbm.at[idx_v.at[c]], fbuf, sem)

    def drain(c, buf):
        rbuf = rows_v.at[buf]
        fbuf = fcv0 if buf == 0 else fcv1
        sem = sem0 if buf == 0 else sem1
        pltpu.make_async_copy(emb_hbm.at[idx_v.at[c]], rbuf, sem).wait()
        pltpu.make_async_copy(fc_hbm.at[idx_v.at[c]], fbuf, sem).wait()

    def compute(c, buf):
        rbuf = rows_v.at[buf]
        fbuf = fcv0 if buf == 0 else fcv1

        for g in range(GPC):
            def item_body(b, pv, g=g):
                r0 = g * ROWS_G + b * F
                s = rbuf[r0, :]
                ss = s * s
                for f in range(1, F):
                    e = rbuf[r0 + f, :]
                    s = s + e
                    ss = ss + e * e
                pair = 0.5 * allsum16(s * s - ss)
                return jnp.where(lanes == b, pair, pv)

            pairvec = lax.fori_loop(0, GI, item_body,
                                    jnp.zeros((16,), jnp.float32))

            rowbase = g * ROWS_G + lanes * F
            fcs = jnp.zeros((16,), jnp.float32)
            for f in range(F):
                fcs = fcs + plsc.load_gather(fbuf, [rowbase + f])

            z = pairvec + fcs * w + bias
            outbuf[pl.ds(c * CI + g * GI, GI)] = 1.0 / (1.0 + jnp.exp(-z))

    fire(0, 0)

    def pair_body(p, _):
        c0 = 2 * p
        fire(c0 + 1, 1)
        drain(c0, 0)
        compute(c0, 0)

        @pl.when(c0 + 2 < NCH)
        def _():
            fire(c0 + 2, 0)

        drain(c0 + 1, 1)
        compute(c0 + 1, 1)
        return 0

    lax.fori_loop(0, NCH // 2, pair_body, 0)

    pltpu.sync_copy(outbuf, out_hbm.at[pl.ds(wid * ITEMS_W, ITEMS_W)])


def kernel(x, emb_table, fc_table, lin_w, lin_b):
    x3 = x.reshape(NW, NCH, ROWS_C)
    wb = jnp.zeros((16,), jnp.float32)
    wb = wb.at[0].set(lin_w[0, 0]).at[1].set(lin_b[0])
    out = _fm_sc(x3, emb_table, fc_table.reshape(-1), wb)
    return out.reshape(B, 1)
